# Initial kernel scaffold; baseline (speedup 1.0000x reference)
import jax
import jax.numpy as jnp
from jax.experimental import pallas as pl


def kernel(x_nchw, weight_oihw, bias_o):
    raise NotImplementedError("write your pallas kernel here")


# speedup vs baseline: 1.5592x; 1.5592x over previous
import functools

import jax
import jax.numpy as jnp
from jax.experimental import pallas as pl
from jax.experimental.pallas import tpu as pltpu


def _conv3x3_kernel(x_ref, w_ref, b_ref, o_ref, *, B, Cin, Cout, H, W):
    HW = H * W
    col = jax.lax.broadcasted_iota(jnp.int32, (Cout, HW), 1) % W
    m_left = col != 0
    m_right = col != (W - 1)
    zrow = jnp.zeros((Cin, W), jnp.float32)
    bias = b_ref[...]
    w_all = w_ref[...]
    for b in range(B):
        xb = x_ref[b]
        r0 = jnp.concatenate([zrow, xb[:, :HW - W]], axis=1)
        r2 = jnp.concatenate([xb[:, W:], zrow], axis=1)
        rows = jnp.concatenate([r0, xb, r2], axis=0)
        t = jnp.dot(w_all, rows, preferred_element_type=jnp.float32)
        t0, t1, t2 = t[:Cout], t[Cout:2 * Cout], t[2 * Cout:]
        s0 = jnp.concatenate([t0[:, :1], t0[:, :HW - 1]], axis=1)
        s2 = jnp.concatenate([t2[:, 1:], t2[:, HW - 1:]], axis=1)
        y = (t1 + jnp.where(m_left, s0, 0.0) + jnp.where(m_right, s2, 0.0)
             + bias)
        o_ref[b] = jnp.maximum(y, 0.0)


def _forward(x_nchw, weight_oihw, bias_o, *, batch_tile):
    N, Cin, H, W = x_nchw.shape
    Cout, _, KH, KW = weight_oihw.shape
    HW = H * W
    xf = x_nchw.reshape(N, Cin, HW)
    w_all = jnp.transpose(weight_oihw, (3, 0, 2, 1)).reshape(KW * Cout,
                                                             KH * Cin)
    b_col = bias_o.reshape(Cout, 1)
    B = batch_tile
    grid = (N // B,)
    cost = pl.CostEstimate(
        flops=2 * N * (KW * Cout) * (KH * Cin) * HW,
        transcendentals=0,
        bytes_accessed=(x_nchw.size * 4 + w_all.size * 4 + Cout * 4
                        + N * Cout * HW * 4),
    )
    out = pl.pallas_call(
        functools.partial(_conv3x3_kernel, B=B, Cin=Cin, Cout=Cout, H=H, W=W),
        out_shape=jax.ShapeDtypeStruct((N, Cout, HW), jnp.float32),
        grid=grid,
        in_specs=[
            pl.BlockSpec((B, Cin, HW), lambda n: (n, 0, 0)),
            pl.BlockSpec((KW * Cout, KH * Cin), lambda n: (0, 0)),
            pl.BlockSpec((Cout, 1), lambda n: (0, 0)),
        ],
        out_specs=pl.BlockSpec((B, Cout, HW), lambda n: (n, 0, 0)),
        compiler_params=pltpu.CompilerParams(
            dimension_semantics=("parallel",)),
        cost_estimate=cost,
    )(xf, w_all, b_col)
    return out.reshape(N, Cout, H, W)


def kernel(x_nchw, weight_oihw, bias_o):
    return _forward(x_nchw, weight_oihw, bias_o, batch_tile=4)


# speedup vs baseline: 3.5608x; 2.2838x over previous
import functools

import jax
import jax.numpy as jnp
from jax.experimental import pallas as pl
from jax.experimental.pallas import tpu as pltpu


def _conv3x3_kernel(x_ref, w_ref, b_ref, o_ref, *, B, Cin, Cout, H, W):
    HW = H * W
    col = jax.lax.broadcasted_iota(jnp.int32, (Cout, HW), 1) % W
    m_left = col != 0
    m_right = col != (W - 1)
    zrow = jnp.zeros((Cin, W), jnp.float32)
    bias = b_ref[...]
    w_all = w_ref[...]
    for b in range(B):
        xb = x_ref[b].reshape(Cin, HW)
        r0 = jnp.concatenate([zrow, xb[:, :HW - W]], axis=1)
        r2 = jnp.concatenate([xb[:, W:], zrow], axis=1)
        rows = jnp.concatenate([r0, xb, r2], axis=0)
        t = jnp.dot(w_all, rows, preferred_element_type=jnp.float32)
        t0, t1, t2 = t[:Cout], t[Cout:2 * Cout], t[2 * Cout:]
        s0 = jnp.concatenate([t0[:, :1], t0[:, :HW - 1]], axis=1)
        s2 = jnp.concatenate([t2[:, 1:], t2[:, HW - 1:]], axis=1)
        y = (t1 + jnp.where(m_left, s0, 0.0) + jnp.where(m_right, s2, 0.0)
             + bias)
        o_ref[b] = jnp.maximum(y, 0.0).reshape(Cout, H, W)


def _forward(x_nchw, weight_oihw, bias_o, *, batch_tile):
    N, Cin, H, W = x_nchw.shape
    Cout, _, KH, KW = weight_oihw.shape
    HW = H * W
    w_all = jnp.transpose(weight_oihw, (3, 0, 2, 1)).reshape(KW * Cout,
                                                             KH * Cin)
    b_col = bias_o.reshape(Cout, 1)
    B = batch_tile
    grid = (N // B,)
    cost = pl.CostEstimate(
        flops=2 * N * (KW * Cout) * (KH * Cin) * HW,
        transcendentals=0,
        bytes_accessed=(x_nchw.size * 4 + w_all.size * 4 + Cout * 4
                        + N * Cout * HW * 4),
    )
    out = pl.pallas_call(
        functools.partial(_conv3x3_kernel, B=B, Cin=Cin, Cout=Cout, H=H, W=W),
        out_shape=jax.ShapeDtypeStruct((N, Cout, H, W), jnp.float32),
        grid=grid,
        in_specs=[
            pl.BlockSpec((B, Cin, H, W), lambda n: (n, 0, 0, 0)),
            pl.BlockSpec((KW * Cout, KH * Cin), lambda n: (0, 0)),
            pl.BlockSpec((Cout, 1), lambda n: (0, 0)),
        ],
        out_specs=pl.BlockSpec((B, Cout, H, W), lambda n: (n, 0, 0, 0)),
        compiler_params=pltpu.CompilerParams(
            dimension_semantics=("parallel",)),
        cost_estimate=cost,
    )(x_nchw, w_all, b_col)
    return out


def kernel(x_nchw, weight_oihw, bias_o):
    return _forward(x_nchw, weight_oihw, bias_o, batch_tile=4)


# speedup vs baseline: 3.7993x; 1.0670x over previous
import functools

import jax
import jax.numpy as jnp
from jax.experimental import pallas as pl
from jax.experimental.pallas import tpu as pltpu


def _conv3x3_kernel(x_ref, w_ref, b_ref, o_ref, *, B, Cin, Cout, H, W):
    HW = H * W
    col = jax.lax.broadcasted_iota(jnp.int32, (Cout, HW), 1) % W
    m_left = col != 0
    m_right = col != (W - 1)
    zrow = jnp.zeros((Cin, W), jnp.float32)
    bias = b_ref[...]
    w_all = w_ref[...]
    for b in range(B):
        xb = x_ref[b].reshape(Cin, HW)
        r0 = jnp.concatenate([zrow, xb[:, :HW - W]], axis=1)
        r2 = jnp.concatenate([xb[:, W:], zrow], axis=1)
        rows = jnp.concatenate([r0, xb, r2], axis=0)
        t = jnp.dot(w_all, rows, preferred_element_type=jnp.float32)
        t0, t1, t2 = t[:Cout], t[Cout:2 * Cout], t[2 * Cout:]
        s0 = jnp.concatenate([t0[:, :1], t0[:, :HW - 1]], axis=1)
        s2 = jnp.concatenate([t2[:, 1:], t2[:, HW - 1:]], axis=1)
        y = (t1 + jnp.where(m_left, s0, 0.0) + jnp.where(m_right, s2, 0.0)
             + bias)
        o_ref[b] = jnp.maximum(y, 0.0).reshape(Cout, H, W)


def _forward(x_nchw, weight_oihw, bias_o, *, batch_tile):
    N, Cin, H, W = x_nchw.shape
    Cout, _, KH, KW = weight_oihw.shape
    HW = H * W
    w_all = jnp.transpose(weight_oihw, (3, 0, 2, 1)).reshape(KW * Cout,
                                                             KH * Cin)
    b_col = bias_o.reshape(Cout, 1)
    B = batch_tile
    grid = (N // B,)
    cost = pl.CostEstimate(
        flops=2 * N * (KW * Cout) * (KH * Cin) * HW,
        transcendentals=0,
        bytes_accessed=(x_nchw.size * 4 + w_all.size * 4 + Cout * 4
                        + N * Cout * HW * 4),
    )
    out = pl.pallas_call(
        functools.partial(_conv3x3_kernel, B=B, Cin=Cin, Cout=Cout, H=H, W=W),
        out_shape=jax.ShapeDtypeStruct((N, Cout, H, W), jnp.float32),
        grid=grid,
        in_specs=[
            pl.BlockSpec((B, Cin, H, W), lambda n: (n, 0, 0, 0)),
            pl.BlockSpec((KW * Cout, KH * Cin), lambda n: (0, 0)),
            pl.BlockSpec((Cout, 1), lambda n: (0, 0)),
        ],
        out_specs=pl.BlockSpec((B, Cout, H, W), lambda n: (n, 0, 0, 0)),
        compiler_params=pltpu.CompilerParams(
            dimension_semantics=("parallel",)),
        cost_estimate=cost,
    )(x_nchw, w_all, b_col)
    return out


def kernel(x_nchw, weight_oihw, bias_o):
    return _forward(x_nchw, weight_oihw, bias_o, batch_tile=8)


# speedup vs baseline: 3.8310x; 1.0083x over previous
import functools

import jax
import jax.numpy as jnp
from jax.experimental import pallas as pl
from jax.experimental.pallas import tpu as pltpu


def _conv3x3_kernel(x_ref, w_ref, b_ref, o_ref, *, B, Cin, Cout, H, W):
    HW = H * W
    col = jax.lax.broadcasted_iota(jnp.int32, (Cout, HW), 1) % W
    m_left = col != 0
    m_right = col != (W - 1)
    zrow = jnp.zeros((Cin, W), jnp.float32)
    bias = b_ref[...]
    w_all = w_ref[...]
    for b in range(B):
        xb = x_ref[b].reshape(Cin, HW)
        r0 = jnp.concatenate([zrow, xb[:, :HW - W]], axis=1)
        r2 = jnp.concatenate([xb[:, W:], zrow], axis=1)
        rows = jnp.concatenate([r0, xb, r2], axis=0)
        t = jnp.dot(w_all, rows, preferred_element_type=jnp.float32)
        t0, t1, t2 = t[:Cout], t[Cout:2 * Cout], t[2 * Cout:]
        s0 = jnp.concatenate([t0[:, :1], t0[:, :HW - 1]], axis=1)
        s2 = jnp.concatenate([t2[:, 1:], t2[:, HW - 1:]], axis=1)
        y = (t1 + jnp.where(m_left, s0, 0.0) + jnp.where(m_right, s2, 0.0)
             + bias)
        o_ref[b] = jnp.maximum(y, 0.0).reshape(Cout, H, W)


def _forward(x_nchw, weight_oihw, bias_o, *, batch_tile):
    N, Cin, H, W = x_nchw.shape
    Cout, _, KH, KW = weight_oihw.shape
    HW = H * W
    w_all = jnp.transpose(weight_oihw, (3, 0, 2, 1)).reshape(KW * Cout,
                                                             KH * Cin)
    b_col = bias_o.reshape(Cout, 1)
    B = batch_tile
    grid = (N // B,)
    cost = pl.CostEstimate(
        flops=2 * N * (KW * Cout) * (KH * Cin) * HW,
        transcendentals=0,
        bytes_accessed=(x_nchw.size * 4 + w_all.size * 4 + Cout * 4
                        + N * Cout * HW * 4),
    )
    out = pl.pallas_call(
        functools.partial(_conv3x3_kernel, B=B, Cin=Cin, Cout=Cout, H=H, W=W),
        out_shape=jax.ShapeDtypeStruct((N, Cout, H, W), jnp.float32),
        grid=grid,
        in_specs=[
            pl.BlockSpec((B, Cin, H, W), lambda n: (n, 0, 0, 0)),
            pl.BlockSpec((KW * Cout, KH * Cin), lambda n: (0, 0)),
            pl.BlockSpec((Cout, 1), lambda n: (0, 0)),
        ],
        out_specs=pl.BlockSpec((B, Cout, H, W), lambda n: (n, 0, 0, 0)),
        compiler_params=pltpu.CompilerParams(
            dimension_semantics=("parallel",)),
        cost_estimate=cost,
    )(x_nchw, w_all, b_col)
    return out


def kernel(x_nchw, weight_oihw, bias_o):
    return _forward(x_nchw, weight_oihw, bias_o, batch_tile=16)


# speedup vs baseline: 4.1949x; 1.0950x over previous
import functools

import jax
import jax.numpy as jnp
from jax.experimental import pallas as pl
from jax.experimental.pallas import tpu as pltpu


def _conv3x3_kernel(x_ref, w_ref, b_ref, o_ref, *, B, Cin, Cout, H, W):
    HW = H * W
    col = jax.lax.broadcasted_iota(jnp.int32, (Cout, HW), 1) % W
    m_left = (col != 0).astype(jnp.float32)
    m_right = (col != (W - 1)).astype(jnp.float32)
    zrow = jnp.zeros((Cin, W), jnp.bfloat16)
    bias = b_ref[...]
    w_all = w_ref[...]
    for b in range(B):
        xb = x_ref[b].reshape(Cin, HW).astype(jnp.bfloat16)
        r0 = jnp.concatenate([zrow, xb[:, :HW - W]], axis=1)
        r2 = jnp.concatenate([xb[:, W:], zrow], axis=1)
        rows = jnp.concatenate([r0, xb, r2], axis=0)
        t = jnp.dot(w_all, rows, preferred_element_type=jnp.float32)
        t0, t1, t2 = t[:Cout], t[Cout:2 * Cout], t[2 * Cout:]
        s0 = jnp.concatenate([t0[:, :1], t0[:, :HW - 1]], axis=1)
        s2 = jnp.concatenate([t2[:, 1:], t2[:, HW - 1:]], axis=1)
        y = t1 + m_left * s0 + m_right * s2 + bias
        o_ref[b] = jnp.maximum(y, 0.0).reshape(Cout, H, W)


def _forward(x_nchw, weight_oihw, bias_o, *, batch_tile):
    N, Cin, H, W = x_nchw.shape
    Cout, _, KH, KW = weight_oihw.shape
    HW = H * W
    w_all = jnp.transpose(weight_oihw, (3, 0, 2, 1)).reshape(
        KW * Cout, KH * Cin).astype(jnp.bfloat16)
    b_col = bias_o.reshape(Cout, 1)
    B = batch_tile
    grid = (N // B,)
    cost = pl.CostEstimate(
        flops=2 * N * (KW * Cout) * (KH * Cin) * HW,
        transcendentals=0,
        bytes_accessed=(x_nchw.size * 4 + w_all.size * 4 + Cout * 4
                        + N * Cout * HW * 4),
    )
    out = pl.pallas_call(
        functools.partial(_conv3x3_kernel, B=B, Cin=Cin, Cout=Cout, H=H, W=W),
        out_shape=jax.ShapeDtypeStruct((N, Cout, H, W), jnp.float32),
        grid=grid,
        in_specs=[
            pl.BlockSpec((B, Cin, H, W), lambda n: (n, 0, 0, 0)),
            pl.BlockSpec((KW * Cout, KH * Cin), lambda n: (0, 0)),
            pl.BlockSpec((Cout, 1), lambda n: (0, 0)),
        ],
        out_specs=pl.BlockSpec((B, Cout, H, W), lambda n: (n, 0, 0, 0)),
        compiler_params=pltpu.CompilerParams(
            dimension_semantics=("parallel",)),
        cost_estimate=cost,
    )(x_nchw, w_all, b_col)
    return out


def kernel(x_nchw, weight_oihw, bias_o):
    return _forward(x_nchw, weight_oihw, bias_o, batch_tile=16)
